# Initial kernel scaffold; baseline (speedup 1.0000x reference)
import jax
import jax.numpy as jnp
from jax.experimental import pallas as pl


def kernel(input_feature, x, coor, Wq, Wk, gq, bq, gk, bk):
    raise NotImplementedError("write your pallas kernel here")


# speedup vs baseline: 21.2272x; 21.2272x over previous
import functools

import jax
import jax.numpy as jnp
from jax import lax
from jax.experimental import pallas as pl
from jax.experimental.pallas import tpu as pltpu
from jax.experimental.pallas import tpu_sc as plsc

_DIM = 256
_NPOINT = 64
_R2 = 16.0
_NS = 8
_B = 8
_N = 16384
_BIG = 1 << 30


def _fps_body(coorT_ref, ifT_ref, ids_ref, scx_ref, scy_ref, scz_ref,
              sifx_ref, sify_ref, sifz_ref):
    cx = coorT_ref[:, 0, :]
    cy = coorT_ref[:, 1, :]
    cz = coorT_ref[:, 2, :]
    fx = ifT_ref[:, 0, :]
    fy = ifT_ref[:, 1, :]
    fz = ifT_ref[:, 2, :]
    li = lax.broadcasted_iota(jnp.int32, (_B, _N), 1)
    li64 = lax.broadcasted_iota(jnp.int32, (_B, _NPOINT), 1)
    boff = lax.broadcasted_iota(jnp.int32, (_B, 1), 0) * _N

    def step(i, carry):
        dists, far, ids, sx, sy, sz, ix, iy, iz = carry
        m = li == far
        centx = jnp.sum(jnp.where(m, cx, 0.0), axis=1, keepdims=True)
        centy = jnp.sum(jnp.where(m, cy, 0.0), axis=1, keepdims=True)
        centz = jnp.sum(jnp.where(m, cz, 0.0), axis=1, keepdims=True)
        cifx = jnp.sum(jnp.where(m, fx, 0.0), axis=1, keepdims=True)
        cify = jnp.sum(jnp.where(m, fy, 0.0), axis=1, keepdims=True)
        cifz = jnp.sum(jnp.where(m, fz, 0.0), axis=1, keepdims=True)
        sel = li64 == i
        ids = jnp.where(sel, far + boff, ids)
        sx = jnp.where(sel, centx, sx)
        sy = jnp.where(sel, centy, sy)
        sz = jnp.where(sel, centz, sz)
        ix = jnp.where(sel, cifx, ix)
        iy = jnp.where(sel, cify, iy)
        iz = jnp.where(sel, cifz, iz)
        dx = cx - centx
        dy = cy - centy
        dz = cz - centz
        d = (dx * dx + dy * dy) + dz * dz
        dists = jnp.minimum(dists, d)
        mx = jnp.max(dists, axis=1, keepdims=True)
        cand = jnp.where(dists == mx, li, _N)
        far = jnp.min(cand, axis=1, keepdims=True)
        return dists, far, ids, sx, sy, sz, ix, iy, iz

    init = (
        jnp.full((_B, _N), 1e10, jnp.float32),
        jnp.zeros((_B, 1), jnp.int32),
        jnp.zeros((_B, _NPOINT), jnp.int32),
        jnp.zeros((_B, _NPOINT), jnp.float32),
        jnp.zeros((_B, _NPOINT), jnp.float32),
        jnp.zeros((_B, _NPOINT), jnp.float32),
        jnp.zeros((_B, _NPOINT), jnp.float32),
        jnp.zeros((_B, _NPOINT), jnp.float32),
        jnp.zeros((_B, _NPOINT), jnp.float32),
    )
    out = lax.fori_loop(0, _NPOINT, step, init)
    _, _, ids, sx, sy, sz, ix, iy, iz = out
    ids_ref[...] = ids
    scx_ref[...] = sx
    scy_ref[...] = sy
    scz_ref[...] = sz
    sifx_ref[...] = ix
    sify_ref[...] = iy
    sifz_ref[...] = iz


def _run_fps(coorT, ifT):
    shape = jax.ShapeDtypeStruct((_B, _NPOINT), jnp.float32)
    return pl.pallas_call(
        _fps_body,
        out_shape=(
            jax.ShapeDtypeStruct((_B, _NPOINT), jnp.int32),
            shape, shape, shape, shape, shape, shape,
        ),
    )(coorT, ifT)


def _bq_body(coorT_ref, ifT_ref, sc_ref, nidx_ref, diffc_ref, meanif_ref):
    b = pl.program_id(0)
    cxr = coorT_ref[0, 0:1, :]
    cyr = coorT_ref[0, 1:2, :]
    czr = coorT_ref[0, 2:3, :]
    scx = sc_ref[0, :, 0:1]
    scy = sc_ref[0, :, 1:2]
    scz = sc_ref[0, :, 2:3]
    dx = scx - cxr
    dy = scy - cyr
    dz = scz - czr
    d2 = (dx * dx + dy * dy) + dz * dz
    mask = d2 < _R2
    li = lax.broadcasted_iota(jnp.int32, (_NPOINT, _N), 1)
    cnt = jnp.sum(mask.astype(jnp.int32), axis=1, keepdims=True)
    mi = jnp.where(mask, li, _BIG)
    idxs = []
    for j in range(_NS):
        mn = jnp.min(mi, axis=1, keepdims=True)
        idxs.append(mn)
        if j + 1 < _NS:
            mi = jnp.where(mi == mn, _BIG, mi)
    first = jnp.where(cnt > 0, idxs[0], 0)
    goff = b * _N
    w = jnp.zeros((_NPOINT, _N), jnp.float32)
    for j in range(_NS):
        idx_j = jnp.where(j < cnt, idxs[j], first)
        nidx_ref[0, :, j:j + 1] = idx_j + goff
        w = w + (li == idx_j).astype(jnp.float32)
    fxr = ifT_ref[0, 0:1, :]
    fyr = ifT_ref[0, 1:2, :]
    fzr = ifT_ref[0, 2:3, :]
    eighth = jnp.float32(1.0 / _NS)
    mcx = jnp.sum(w * cxr, axis=1, keepdims=True) * eighth
    mcy = jnp.sum(w * cyr, axis=1, keepdims=True) * eighth
    mcz = jnp.sum(w * czr, axis=1, keepdims=True) * eighth
    diffc_ref[0, :, 0:1] = mcx - scx
    diffc_ref[0, :, 1:2] = mcy - scy
    diffc_ref[0, :, 2:3] = mcz - scz
    meanif_ref[0, :, 0:1] = jnp.sum(w * fxr, axis=1, keepdims=True) * eighth
    meanif_ref[0, :, 1:2] = jnp.sum(w * fyr, axis=1, keepdims=True) * eighth
    meanif_ref[0, :, 2:3] = jnp.sum(w * fzr, axis=1, keepdims=True) * eighth


def _run_bq(coorT, ifT, sample_coor):
    return pl.pallas_call(
        _bq_body,
        grid=(_B,),
        in_specs=[
            pl.BlockSpec((1, 3, _N), lambda b: (b, 0, 0)),
            pl.BlockSpec((1, 3, _N), lambda b: (b, 0, 0)),
            pl.BlockSpec((1, _NPOINT, 3), lambda b: (b, 0, 0)),
        ],
        out_specs=[
            pl.BlockSpec((1, _NPOINT, _NS), lambda b: (b, 0, 0)),
            pl.BlockSpec((1, _NPOINT, 3), lambda b: (b, 0, 0)),
            pl.BlockSpec((1, _NPOINT, 3), lambda b: (b, 0, 0)),
        ],
        out_shape=[
            jax.ShapeDtypeStruct((_B, _NPOINT, _NS), jnp.int32),
            jax.ShapeDtypeStruct((_B, _NPOINT, 3), jnp.float32),
            jax.ShapeDtypeStruct((_B, _NPOINT, 3), jnp.float32),
        ],
    )(coorT, ifT, sample_coor)


_NWORK = 32
_S_PER_W = (_B * _NPOINT) // _NWORK
_ROWS_PER_W = _S_PER_W * _NS


def _sc_body(x_hbm, sidx_hbm, nidx_hbm, sx_out, gx_out,
             sidx_v, nidx_v, srows, nrows, pooled, sem1, sem2):
    wid = lax.axis_index("s") * 2 + lax.axis_index("c")
    sb = wid * _S_PER_W
    nb = wid * _ROWS_PER_W
    pltpu.sync_copy(sidx_hbm.at[pl.ds(sb, _S_PER_W)], sidx_v)
    pltpu.sync_copy(nidx_hbm.at[pl.ds(nb, _ROWS_PER_W)], nidx_v)
    c1 = pltpu.async_copy(x_hbm.at[nidx_v], nrows, sem1)
    c2 = pltpu.async_copy(x_hbm.at[sidx_v], srows, sem2)
    c1.wait()
    c2.wait()

    def pool_one(s, carry):
        base = s * _NS
        for c in range(_DIM // 16):
            sl = pl.ds(c * 16, 16)
            m = nrows[base, sl]
            for r in range(1, _NS):
                m = jnp.maximum(m, nrows[base + r, sl])
            pooled[s, sl] = m
        return carry

    lax.fori_loop(0, _S_PER_W, pool_one, 0)
    pltpu.sync_copy(pooled, gx_out.at[pl.ds(sb, _S_PER_W)])
    pltpu.sync_copy(srows, sx_out.at[pl.ds(sb, _S_PER_W)])


def _run_gather_pool(x2d, sidx, nidx):
    nsamp = _B * _NPOINT
    mesh = plsc.VectorSubcoreMesh(core_axis_name="c", subcore_axis_name="s")
    f = pl.kernel(
        _sc_body,
        out_type=(
            jax.ShapeDtypeStruct((nsamp, _DIM), jnp.float32),
            jax.ShapeDtypeStruct((nsamp, _DIM), jnp.float32),
        ),
        mesh=mesh,
        scratch_types=[
            pltpu.VMEM((_S_PER_W,), jnp.int32),
            pltpu.VMEM((_ROWS_PER_W,), jnp.int32),
            pltpu.VMEM((_S_PER_W, _DIM), jnp.float32),
            pltpu.VMEM((_ROWS_PER_W, _DIM), jnp.float32),
            pltpu.VMEM((_S_PER_W, _DIM), jnp.float32),
            pltpu.SemaphoreType.DMA,
            pltpu.SemaphoreType.DMA,
        ],
    )
    return f(x2d, sidx, nidx)


def _ln(v, g, bvec):
    mu = jnp.mean(v, axis=-1, keepdims=True)
    var = jnp.mean((v - mu) ** 2, axis=-1, keepdims=True)
    return (v - mu) / jnp.sqrt(var + 1e-5) * g + bvec


def _attn_body(sx_ref, gx_ref, vc_ref, vi_ref, sc_ref, sif_ref,
               wqt_ref, wkt_ref, gq_ref, bq_ref, gk_ref, bk_ref,
               outx_ref, outc_ref, outi_ref):
    sxb = sx_ref[0]
    gxb = gx_ref[0]
    x2 = gxb - sxb
    nk = _ln(sxb, gk_ref[...], bk_ref[...])
    nq = _ln(x2, gq_ref[...], bq_ref[...])
    qh = jnp.dot(nq, wqt_ref[...], preferred_element_type=jnp.float32)
    kh = jnp.dot(nk, wkt_ref[...], preferred_element_type=jnp.float32)
    attn = lax.dot_general(qh, kh, (((1,), (1,)), ((), ())),
                           preferred_element_type=jnp.float32)
    mx = jnp.max(attn, axis=-1, keepdims=True)
    e = jnp.exp(attn - mx)
    p = e / jnp.sum(e, axis=-1, keepdims=True)
    c2 = jnp.dot(p, vc_ref[0], preferred_element_type=jnp.float32)
    i2 = jnp.dot(p, vi_ref[0], preferred_element_type=jnp.float32)
    outx_ref[0] = sxb + x2
    outc_ref[0] = sc_ref[0] + c2
    outi_ref[0] = sif_ref[0] + i2


def _run_attn(sx, gx, v_c, v_i, sample_coor, sif, WqT, WkT, gq, bq, gk, bk):
    spec64 = pl.BlockSpec((1, _NPOINT, _DIM), lambda b: (b, 0, 0))
    spec3 = pl.BlockSpec((1, _NPOINT, 3), lambda b: (b, 0, 0))
    specw = pl.BlockSpec((_DIM, _DIM), lambda b: (0, 0))
    specv = pl.BlockSpec((1, _DIM), lambda b: (0, 0))
    return pl.pallas_call(
        _attn_body,
        grid=(_B,),
        in_specs=[spec64, spec64, spec3, spec3, spec3, spec3,
                  specw, specw, specv, specv, specv, specv],
        out_specs=[spec64, spec3, spec3],
        out_shape=[
            jax.ShapeDtypeStruct((_B, _NPOINT, _DIM), jnp.float32),
            jax.ShapeDtypeStruct((_B, _NPOINT, 3), jnp.float32),
            jax.ShapeDtypeStruct((_B, _NPOINT, 3), jnp.float32),
        ],
    )(sx, gx, v_c, v_i, sample_coor, sif, WqT, WkT, gq, bq, gk, bk)


def kernel(input_feature, x, coor, Wq, Wk, gq, bq, gk, bk):
    coorT = jnp.transpose(coor, (0, 2, 1))
    ifT = jnp.transpose(input_feature, (0, 2, 1))
    ids_g, scx, scy, scz, sifx, sify, sifz = _run_fps(coorT, ifT)
    sample_coor = jnp.stack([scx, scy, scz], axis=-1)
    sif = jnp.stack([sifx, sify, sifz], axis=-1)
    nidx, diffc, meanif = _run_bq(coorT, ifT, sample_coor)
    sx, gx = _run_gather_pool(
        x.reshape(_B * _N, _DIM), ids_g.reshape(-1), nidx.reshape(-1))
    v_c = jnp.transpose(diffc, (0, 2, 1)).reshape(_B, _NPOINT, 3)
    v_i = jnp.transpose(meanif, (0, 2, 1)).reshape(_B, _NPOINT, 3)
    return _run_attn(
        sx.reshape(_B, _NPOINT, _DIM), gx.reshape(_B, _NPOINT, _DIM),
        v_c, v_i, sample_coor, sif, Wq.T, Wk.T,
        gq.reshape(1, _DIM), bq.reshape(1, _DIM),
        gk.reshape(1, _DIM), bk.reshape(1, _DIM))
